# baseline (speedup 1.0000x reference)
import functools

import jax
import jax.numpy as jnp
from jax import lax
from jax.experimental import pallas as pl
from jax.experimental.pallas import tpu as pltpu
from jax.experimental.pallas import tpu_sc as plsc

NUM_EMBEDDINGS = 100000
EMBEDDING_DIM = 128
BATCH = 4096
HIST_LEN = 50

NC = 2
NS = 16
NW = NC * NS

B_PER_W = BATCH // NW
TG = 1
GROUPS = HIST_LEN // TG
NBUF = 4


def _make_kernel():
    mesh = plsc.VectorSubcoreMesh(
        core_axis_name="c", subcore_axis_name="s",
        num_cores=NC, num_subcores=NS)

    @functools.partial(
        pl.kernel,
        out_type=jax.ShapeDtypeStruct((HIST_LEN, BATCH, EMBEDDING_DIM),
                                      jnp.float32),
        mesh=mesh,
        scratch_types=(
            [pltpu.VMEM((1, HIST_LEN, B_PER_W), jnp.int32)]
            + [pltpu.VMEM((TG, B_PER_W, EMBEDDING_DIM), jnp.float32)
               for _ in range(NBUF)]
            + [pltpu.SemaphoreType.DMA for _ in range(2 * NBUF)]
        ),
    )
    def gather_kernel(idx_hbm, table_hbm, out_hbm, idx_v, *scratch):
        bufs = scratch[:NBUF]
        gsem = scratch[NBUF:2 * NBUF]
        psem = scratch[2 * NBUF:]
        wid = lax.axis_index("s") * NC + lax.axis_index("c")
        bcol = wid * B_PER_W
        pltpu.sync_copy(idx_hbm.at[pl.ds(wid, 1)], idx_v)

        def gather_group(grp, b):
            for i in range(TG):
                pltpu.async_copy(
                    table_hbm.at[idx_v.at[0, grp * TG + i]],
                    bufs[b].at[i], gsem[b])

        def wait_group(b):
            for i in range(TG):
                pltpu.make_async_copy(
                    table_hbm.at[idx_v.at[0, 0]],
                    bufs[b].at[i], gsem[b]).wait()

        for b in range(NBUF):
            gather_group(b, b)

        def outer(t, carry):
            for b in range(NBUF):
                grp = t * NBUF + b

                @pl.when(grp < GROUPS)
                def _():
                    wait_group(b)
                    pltpu.async_copy(
                        bufs[b],
                        out_hbm.at[pl.ds(grp * TG, TG), pl.ds(bcol, B_PER_W)],
                        psem[b])
                    gn = grp + NBUF

                    @pl.when(gn < GROUPS)
                    def _():
                        pltpu.make_async_copy(
                            bufs[b],
                            out_hbm.at[pl.ds(0, TG), pl.ds(bcol, B_PER_W)],
                            psem[b]).wait()
                        gather_group(gn, b)
            return carry

        lax.fori_loop(0, (GROUPS + NBUF - 1) // NBUF, outer, 0)
        for b in range(NBUF):
            pltpu.make_async_copy(
                bufs[b],
                out_hbm.at[pl.ds(0, TG), pl.ds(bcol, B_PER_W)],
                psem[b]).wait()

    return gather_kernel


_gather = _make_kernel()


def kernel(token_ids, embedding_mat):
    idx = jnp.transpose(
        jnp.reshape(token_ids.astype(jnp.int32), (NW, B_PER_W, HIST_LEN)),
        (0, 2, 1))
    out = _gather(idx, embedding_mat)
    return jnp.transpose(out, (1, 0, 2))


# speedup vs baseline: 1.0016x; 1.0016x over previous
import functools

import jax
import jax.numpy as jnp
from jax import lax
from jax.experimental import pallas as pl
from jax.experimental.pallas import tpu as pltpu
from jax.experimental.pallas import tpu_sc as plsc

NUM_EMBEDDINGS = 100000
EMBEDDING_DIM = 128
BATCH = 4096
HIST_LEN = 50

NC = 2
NS = 16
NW = NC * NS

B_PER_W = BATCH // NW
TG = 2
GROUPS = HIST_LEN // TG
NBUF = 3


def _make_kernel():
    mesh = plsc.VectorSubcoreMesh(
        core_axis_name="c", subcore_axis_name="s",
        num_cores=NC, num_subcores=NS)

    @functools.partial(
        pl.kernel,
        out_type=jax.ShapeDtypeStruct((HIST_LEN, BATCH, EMBEDDING_DIM),
                                      jnp.float32),
        mesh=mesh,
        scratch_types=(
            [pltpu.VMEM((1, HIST_LEN, B_PER_W), jnp.int32)]
            + [pltpu.VMEM((TG, B_PER_W, EMBEDDING_DIM), jnp.float32)
               for _ in range(NBUF)]
            + [pltpu.SemaphoreType.DMA for _ in range(2 * NBUF)]
        ),
    )
    def gather_kernel(idx_hbm, table_hbm, out_hbm, idx_v, *scratch):
        bufs = scratch[:NBUF]
        gsem = scratch[NBUF:2 * NBUF]
        psem = scratch[2 * NBUF:]
        wid = lax.axis_index("s") * NC + lax.axis_index("c")
        bcol = wid * B_PER_W
        pltpu.sync_copy(idx_hbm.at[pl.ds(wid, 1)], idx_v)

        def gather_group(grp, b):
            for i in range(TG):
                pltpu.async_copy(
                    table_hbm.at[idx_v.at[0, grp * TG + i]],
                    bufs[b].at[i], gsem[b])

        def wait_group(b):
            for i in range(TG):
                pltpu.make_async_copy(
                    table_hbm.at[idx_v.at[0, 0]],
                    bufs[b].at[i], gsem[b]).wait()

        for b in range(NBUF):
            gather_group(b, b)

        def outer(t, carry):
            for b in range(NBUF):
                grp = t * NBUF + b

                @pl.when(grp < GROUPS)
                def _():
                    wait_group(b)
                    pltpu.async_copy(
                        bufs[b],
                        out_hbm.at[pl.ds(grp * TG, TG), pl.ds(bcol, B_PER_W)],
                        psem[b])
                    gn = grp + NBUF

                    @pl.when(gn < GROUPS)
                    def _():
                        pltpu.make_async_copy(
                            bufs[b],
                            out_hbm.at[pl.ds(0, TG), pl.ds(bcol, B_PER_W)],
                            psem[b]).wait()
                        gather_group(gn, b)
            return carry

        lax.fori_loop(0, (GROUPS + NBUF - 1) // NBUF, outer, 0)
        for b in range(NBUF):
            pltpu.make_async_copy(
                bufs[b],
                out_hbm.at[pl.ds(0, TG), pl.ds(bcol, B_PER_W)],
                psem[b]).wait()

    return gather_kernel


_gather = _make_kernel()


def kernel(token_ids, embedding_mat):
    idx = jnp.transpose(
        jnp.reshape(token_ids.astype(jnp.int32), (NW, B_PER_W, HIST_LEN)),
        (0, 2, 1))
    out = _gather(idx, embedding_mat)
    return jnp.transpose(out, (1, 0, 2))


# speedup vs baseline: 1.0101x; 1.0085x over previous
import functools

import jax
import jax.numpy as jnp
from jax import lax
from jax.experimental import pallas as pl
from jax.experimental.pallas import tpu as pltpu
from jax.experimental.pallas import tpu_sc as plsc

NUM_EMBEDDINGS = 100000
EMBEDDING_DIM = 128
BATCH = 4096
HIST_LEN = 50

NC = 2
NS = 16
NW = NC * NS

B_PER_W = BATCH // NW
TG = 2
GROUPS = HIST_LEN // TG
NBUF = 2


def _make_kernel():
    mesh = plsc.VectorSubcoreMesh(
        core_axis_name="c", subcore_axis_name="s",
        num_cores=NC, num_subcores=NS)

    @functools.partial(
        pl.kernel,
        out_type=jax.ShapeDtypeStruct((HIST_LEN, BATCH, EMBEDDING_DIM),
                                      jnp.float32),
        mesh=mesh,
        scratch_types=(
            [pltpu.VMEM((HIST_LEN, B_PER_W), jnp.int32)]
            + [pltpu.VMEM((TG, B_PER_W, EMBEDDING_DIM), jnp.float32)
               for _ in range(NBUF)]
            + [pltpu.SemaphoreType.DMA for _ in range(2 * NBUF)]
        ),
    )
    def gather_kernel(idx_hbm, table_hbm, out_hbm, idx_v, *scratch):
        bufs = scratch[:NBUF]
        gsem = scratch[NBUF:2 * NBUF]
        psem = scratch[2 * NBUF:]
        wid = lax.axis_index("s") * NC + lax.axis_index("c")
        bcol = wid * B_PER_W
        pltpu.sync_copy(idx_hbm.at[:, pl.ds(bcol, B_PER_W)], idx_v)

        def gather_group(grp, b):
            for i in range(TG):
                pltpu.async_copy(
                    table_hbm.at[idx_v.at[grp * TG + i]],
                    bufs[b].at[i], gsem[b])

        def wait_group(b):
            for i in range(TG):
                pltpu.make_async_copy(
                    table_hbm.at[idx_v.at[0]],
                    bufs[b].at[i], gsem[b]).wait()

        for b in range(NBUF):
            gather_group(b, b)

        def outer(t, carry):
            for b in range(NBUF):
                grp = t * NBUF + b

                @pl.when(grp < GROUPS)
                def _():
                    wait_group(b)
                    pltpu.async_copy(
                        bufs[b],
                        out_hbm.at[pl.ds(grp * TG, TG), pl.ds(bcol, B_PER_W)],
                        psem[b])
                    gn = grp + NBUF

                    @pl.when(gn < GROUPS)
                    def _():
                        pltpu.make_async_copy(
                            bufs[b],
                            out_hbm.at[pl.ds(0, TG), pl.ds(bcol, B_PER_W)],
                            psem[b]).wait()
                        gather_group(gn, b)
            return carry

        lax.fori_loop(0, (GROUPS + NBUF - 1) // NBUF, outer, 0)
        for b in range(NBUF):
            pltpu.make_async_copy(
                bufs[b],
                out_hbm.at[pl.ds(0, TG), pl.ds(bcol, B_PER_W)],
                psem[b]).wait()

    return gather_kernel


_gather = _make_kernel()


def kernel(token_ids, embedding_mat):
    idx_t = jnp.transpose(token_ids.astype(jnp.int32))
    out = _gather(idx_t, embedding_mat)
    return jnp.transpose(out, (1, 0, 2))
